# baseline (speedup 1.0000x reference)
import functools

import jax
import jax.numpy as jnp
from jax import lax
from jax.experimental import pallas as pl
from jax.experimental.pallas import tpu as pltpu
from jax.experimental.pallas import tpu_sc as plsc

BATCH = 16384
LATENT_DIM = 32
NUM_WORKERS = 32
B_PER_W = BATCH // NUM_WORKERS
IDX_CHUNK = 128
N_CHUNKS = B_PER_W // IDX_CHUNK
LANES = 16
N_GROUPS = B_PER_W // LANES


@functools.partial(
    pl.kernel,
    out_type=jax.ShapeDtypeStruct((BATCH,), jnp.float32),
    mesh=plsc.VectorSubcoreMesh(core_axis_name="c", subcore_axis_name="s"),
    compiler_params=pltpu.CompilerParams(
        needs_layout_passes=False, use_tc_tiling_on_sc=False),
    scratch_types=[
        pltpu.VMEM((N_CHUNKS, IDX_CHUNK), jnp.int32),
        pltpu.VMEM((N_CHUNKS, IDX_CHUNK), jnp.int32),
        pltpu.VMEM((B_PER_W, LATENT_DIM), jnp.float32),
        pltpu.VMEM((B_PER_W, LATENT_DIM), jnp.float32),
        pltpu.VMEM((LATENT_DIM + LANES,), jnp.float32),
        pltpu.VMEM((B_PER_W,), jnp.float32),
        pltpu.SemaphoreType.DMA,
    ],
)
def _gmf_sc(uidx_hbm, iidx_hbm, utab_hbm, itab_hbm, params_hbm, out_hbm,
            uidx_v, iidx_v, urows_v, irows_v, params_v, out_v, sem):
    wid = lax.axis_index("s") * 2 + lax.axis_index("c")
    idx_row0 = wid * N_CHUNKS

    pltpu.sync_copy(params_hbm, params_v)
    pltpu.sync_copy(uidx_hbm.at[pl.ds(idx_row0, N_CHUNKS)], uidx_v)
    pltpu.sync_copy(iidx_hbm.at[pl.ds(idx_row0, N_CHUNKS)], iidx_v)

    copies = []
    for j in range(N_CHUNKS):
        rows = pl.ds(j * IDX_CHUNK, IDX_CHUNK)
        copies.append(pltpu.async_copy(utab_hbm.at[uidx_v.at[j]],
                                       urows_v.at[rows], sem))
        copies.append(pltpu.async_copy(itab_hbm.at[iidx_v.at[j]],
                                       irows_v.at[rows], sem))
    for c in copies:
        c.wait()

    lane_iota = lax.iota(jnp.int32, LANES)
    wvec0 = params_v[pl.ds(0, LANES)]
    wvec1 = params_v[pl.ds(LANES, LANES)]
    bias_vec = params_v[pl.ds(LATENT_DIM, LANES)]

    def group_body(g, carry):
        b_idx = g * LANES + lane_iota
        packed = jnp.zeros((LANES,), jnp.float32)
        for k in range(LANES):
            r = g * LANES + k
            u0 = urows_v[r, pl.ds(0, LANES)]
            u1 = urows_v[r, pl.ds(LANES, LANES)]
            i0 = irows_v[r, pl.ds(0, LANES)]
            i1 = irows_v[r, pl.ds(LANES, LANES)]
            t = u0 * i0 * wvec0 + u1 * i1 * wvec1
            s = jnp.sum(t)
            packed = jnp.where(lane_iota == k, s, packed)
        z = packed + bias_vec
        sig = 1.0 / (1.0 + jnp.exp(-z))
        plsc.store_scatter(out_v, [b_idx], sig)
        return carry

    lax.fori_loop(0, N_GROUPS, group_body, 0)
    pltpu.sync_copy(out_v, out_hbm.at[pl.ds(wid * B_PER_W, B_PER_W)])


def kernel(user_indices, item_indices, user_table, item_table, affine_w, affine_b):
    uidx = user_indices.reshape(NUM_WORKERS * N_CHUNKS, IDX_CHUNK)
    iidx = item_indices.reshape(NUM_WORKERS * N_CHUNKS, IDX_CHUNK)
    params = jnp.concatenate(
        [affine_w.reshape(-1),
         jnp.broadcast_to(affine_b.reshape(-1), (LANES,))]).astype(jnp.float32)
    out = _gmf_sc(uidx, iidx, user_table, item_table, params)
    return out.reshape(BATCH, 1)


# speedup vs baseline: 3.8011x; 3.8011x over previous
import functools

import jax
import jax.numpy as jnp
from jax import lax
from jax.experimental import pallas as pl
from jax.experimental.pallas import tpu as pltpu
from jax.experimental.pallas import tpu_sc as plsc

BATCH = 16384
LATENT_DIM = 32
NUM_WORKERS = 32
B_PER_W = BATCH // NUM_WORKERS
LANES = 16
N_GROUPS = B_PER_W // LANES
NSLOT = 4
TILE_L = 128


@functools.partial(
    pl.kernel,
    out_type=jax.ShapeDtypeStruct((BATCH,), jnp.float32),
    mesh=plsc.VectorSubcoreMesh(core_axis_name="c", subcore_axis_name="s"),
    compiler_params=pltpu.CompilerParams(needs_layout_passes=False),
    scratch_types=[
        pltpu.VMEM((B_PER_W,), jnp.int32),
        pltpu.VMEM((B_PER_W,), jnp.int32),
        pltpu.VMEM((NSLOT, LATENT_DIM, TILE_L), jnp.float32),
        pltpu.VMEM((NSLOT, LATENT_DIM, TILE_L), jnp.float32),
        pltpu.VMEM((LATENT_DIM, B_PER_W), jnp.float32),
        pltpu.VMEM((LATENT_DIM, B_PER_W), jnp.float32),
        pltpu.VMEM((LATENT_DIM + 1, LANES), jnp.float32),
        pltpu.VMEM((B_PER_W,), jnp.float32),
        pltpu.SemaphoreType.DMA,
        pltpu.SemaphoreType.DMA,
    ],
)
def _gmf_sc(uidx_hbm, iidx_hbm, utab_hbm, itab_hbm, params_hbm, out_hbm,
            uidx_v, iidx_v, uslot_v, islot_v, ubuf_v, ibuf_v, params_v,
            out_v, sem_u, sem_i):
    wid = lax.axis_index("s") * 2 + lax.axis_index("c")
    base = wid * B_PER_W

    pltpu.sync_copy(params_hbm, params_v)
    pltpu.sync_copy(uidx_hbm.at[pl.ds(base, B_PER_W)], uidx_v)
    pltpu.sync_copy(iidx_hbm.at[pl.ds(base, B_PER_W)], iidx_v)

    d_lo = lax.iota(jnp.int32, LANES)
    d_hi = d_lo + LANES

    def fire(tab_hbm, slot_v, sem, idx_scalar, s):
        vt = lax.shift_right_logical(idx_scalar, 7)
        cols = pl.ds(pl.multiple_of(vt * TILE_L, TILE_L), TILE_L)
        pltpu.async_copy(tab_hbm.at[:, cols], slot_v.at[s], sem)

    def wait_slot(tab_hbm, slot_v, sem, s):
        pltpu.make_async_copy(tab_hbm.at[:, pl.ds(0, TILE_L)],
                              slot_v.at[s], sem).wait()

    def extract(slot_v, buf_v, idx_scalar, s, b):
        voff = lax.bitwise_and(idx_scalar, TILE_L - 1)
        s_vec = jnp.full((LANES,), s, jnp.int32)
        voff_vec = jnp.full((LANES,), voff, jnp.int32)
        b_vec = jnp.full((LANES,), b, jnp.int32)
        v0 = plsc.load_gather(slot_v, [s_vec, d_lo, voff_vec])
        v1 = plsc.load_gather(slot_v, [s_vec, d_hi, voff_vec])
        plsc.store_scatter(buf_v, [d_lo, b_vec], v0)
        plsc.store_scatter(buf_v, [d_hi, b_vec], v1)

    uvec_p = uidx_v[pl.ds(0, LANES)]
    ivec_p = iidx_v[pl.ds(0, LANES)]
    for k in range(NSLOT):
        fire(utab_hbm, uslot_v, sem_u, uvec_p[k], k)
        fire(itab_hbm, islot_v, sem_i, ivec_p[k], k)

    def group_body(g, carry):
        uvec = uidx_v[pl.ds(g * LANES, LANES)]
        ivec = iidx_v[pl.ds(g * LANES, LANES)]
        g1 = jnp.minimum(g + 1, N_GROUPS - 1)
        uvec1 = uidx_v[pl.ds(g1 * LANES, LANES)]
        ivec1 = iidx_v[pl.ds(g1 * LANES, LANES)]
        for k in range(LANES):
            b = g * LANES + k
            s = k % NSLOT
            wait_slot(utab_hbm, uslot_v, sem_u, s)
            extract(uslot_v, ubuf_v, uvec[k], s, b)
            wait_slot(itab_hbm, islot_v, sem_i, s)
            extract(islot_v, ibuf_v, ivec[k], s, b)
            kn = k + NSLOT
            if kn < LANES:
                fire(utab_hbm, uslot_v, sem_u, uvec[kn], s)
                fire(itab_hbm, islot_v, sem_i, ivec[kn], s)
            else:
                @pl.when(g < N_GROUPS - 1)
                def _():
                    fire(utab_hbm, uslot_v, sem_u, uvec1[kn - LANES], s)
                    fire(itab_hbm, islot_v, sem_i, ivec1[kn - LANES], s)
        return carry

    lax.fori_loop(0, N_GROUPS, group_body, 0)

    lane_iota = lax.iota(jnp.int32, LANES)
    wvecs = [params_v[d, pl.ds(0, LANES)] for d in range(LATENT_DIM)]
    bias_vec = params_v[LATENT_DIM, pl.ds(0, LANES)]

    def compute_body(g, carry):
        col = g * LANES
        acc = bias_vec
        for d in range(LATENT_DIM):
            u = ubuf_v[d, pl.ds(col, LANES)]
            i = ibuf_v[d, pl.ds(col, LANES)]
            acc = acc + (u * i) * wvecs[d]
        sig = 1.0 / (1.0 + jnp.exp(-acc))
        plsc.store_scatter(out_v, [col + lane_iota], sig)
        return carry

    lax.fori_loop(0, N_GROUPS, compute_body, 0)
    pltpu.sync_copy(out_v, out_hbm.at[pl.ds(base, B_PER_W)])


def kernel(user_indices, item_indices, user_table, item_table, affine_w, affine_b):
    params = jnp.concatenate(
        [jnp.broadcast_to(affine_w.reshape(LATENT_DIM, 1), (LATENT_DIM, LANES)),
         jnp.broadcast_to(affine_b.reshape(1, 1), (1, LANES))]).astype(jnp.float32)
    out = _gmf_sc(user_indices, item_indices, user_table.T, item_table.T, params)
    return out.reshape(BATCH, 1)


# speedup vs baseline: 4.4039x; 1.1586x over previous
import functools

import jax
import jax.numpy as jnp
from jax import lax
from jax.experimental import pallas as pl
from jax.experimental.pallas import tpu as pltpu
from jax.experimental.pallas import tpu_sc as plsc

BATCH = 16384
LATENT_DIM = 32
NUM_WORKERS = 32
B_PER_W = BATCH // NUM_WORKERS
LANES = 16
N_GROUPS = B_PER_W // LANES
NSLOT = 8
TILE_L = 128


@functools.partial(
    pl.kernel,
    out_type=jax.ShapeDtypeStruct((BATCH,), jnp.float32),
    mesh=plsc.VectorSubcoreMesh(core_axis_name="c", subcore_axis_name="s"),
    compiler_params=pltpu.CompilerParams(needs_layout_passes=False),
    scratch_types=[
        pltpu.VMEM((B_PER_W,), jnp.int32),
        pltpu.VMEM((B_PER_W,), jnp.int32),
        pltpu.VMEM((NSLOT, LATENT_DIM, TILE_L), jnp.float32),
        pltpu.VMEM((NSLOT, LATENT_DIM, TILE_L), jnp.float32),
        pltpu.VMEM((LATENT_DIM, B_PER_W), jnp.float32),
        pltpu.VMEM((LATENT_DIM, B_PER_W), jnp.float32),
        pltpu.VMEM((LATENT_DIM + 1, LANES), jnp.float32),
        pltpu.VMEM((B_PER_W,), jnp.float32),
        pltpu.SemaphoreType.DMA,
        pltpu.SemaphoreType.DMA,
    ],
)
def _gmf_sc(uidx_hbm, iidx_hbm, utab_hbm, itab_hbm, params_hbm, out_hbm,
            uidx_v, iidx_v, uslot_v, islot_v, ubuf_v, ibuf_v, params_v,
            out_v, sem_u, sem_i):
    wid = lax.axis_index("s") * 2 + lax.axis_index("c")
    base = wid * B_PER_W

    pltpu.sync_copy(params_hbm, params_v)
    pltpu.sync_copy(uidx_hbm.at[pl.ds(base, B_PER_W)], uidx_v)
    pltpu.sync_copy(iidx_hbm.at[pl.ds(base, B_PER_W)], iidx_v)

    d_lo = lax.iota(jnp.int32, LANES)
    d_hi = d_lo + LANES

    def fire(tab_hbm, slot_v, sem, idx_scalar, s):
        vt = lax.shift_right_logical(idx_scalar, 7)
        cols = pl.ds(pl.multiple_of(vt * TILE_L, TILE_L), TILE_L)
        pltpu.async_copy(tab_hbm.at[:, cols], slot_v.at[s], sem)

    def wait_slot(tab_hbm, slot_v, sem, s):
        pltpu.make_async_copy(tab_hbm.at[:, pl.ds(0, TILE_L)],
                              slot_v.at[s], sem).wait()

    def extract(slot_v, buf_v, idx_scalar, s, b):
        voff = lax.bitwise_and(idx_scalar, TILE_L - 1)
        s_vec = jnp.full((LANES,), s, jnp.int32)
        voff_vec = jnp.full((LANES,), voff, jnp.int32)
        b_vec = jnp.full((LANES,), b, jnp.int32)
        v0 = plsc.load_gather(slot_v, [s_vec, d_lo, voff_vec])
        v1 = plsc.load_gather(slot_v, [s_vec, d_hi, voff_vec])
        plsc.store_scatter(buf_v, [d_lo, b_vec], v0)
        plsc.store_scatter(buf_v, [d_hi, b_vec], v1)

    uvec_p = uidx_v[pl.ds(0, LANES)]
    ivec_p = iidx_v[pl.ds(0, LANES)]
    for k in range(NSLOT):
        fire(utab_hbm, uslot_v, sem_u, uvec_p[k], k)
        fire(itab_hbm, islot_v, sem_i, ivec_p[k], k)

    lane_iota = lax.iota(jnp.int32, LANES)
    wvecs = [params_v[d, pl.ds(0, LANES)] for d in range(LATENT_DIM)]
    bias_vec = params_v[LATENT_DIM, pl.ds(0, LANES)]

    def group_body(g, carry):
        uvec = uidx_v[pl.ds(g * LANES, LANES)]
        ivec = iidx_v[pl.ds(g * LANES, LANES)]
        g1 = jnp.minimum(g + 1, N_GROUPS - 1)
        uvec1 = uidx_v[pl.ds(g1 * LANES, LANES)]
        ivec1 = iidx_v[pl.ds(g1 * LANES, LANES)]
        for k in range(LANES):
            b = g * LANES + k
            s = k % NSLOT
            kn = k + NSLOT
            wait_slot(utab_hbm, uslot_v, sem_u, s)
            extract(uslot_v, ubuf_v, uvec[k], s, b)
            if kn < LANES:
                fire(utab_hbm, uslot_v, sem_u, uvec[kn], s)
            else:
                @pl.when(g < N_GROUPS - 1)
                def _():
                    fire(utab_hbm, uslot_v, sem_u, uvec1[kn - LANES], s)
            wait_slot(itab_hbm, islot_v, sem_i, s)
            extract(islot_v, ibuf_v, ivec[k], s, b)
            if kn < LANES:
                fire(itab_hbm, islot_v, sem_i, ivec[kn], s)
            else:
                @pl.when(g < N_GROUPS - 1)
                def _():
                    fire(itab_hbm, islot_v, sem_i, ivec1[kn - LANES], s)
        col = g * LANES
        acc = bias_vec
        for d in range(LATENT_DIM):
            u = ubuf_v[d, pl.ds(col, LANES)]
            i = ibuf_v[d, pl.ds(col, LANES)]
            acc = acc + (u * i) * wvecs[d]
        sig = 1.0 / (1.0 + jnp.exp(-acc))
        plsc.store_scatter(out_v, [col + lane_iota], sig)
        return carry

    lax.fori_loop(0, N_GROUPS, group_body, 0)
    pltpu.sync_copy(out_v, out_hbm.at[pl.ds(base, B_PER_W)])


def kernel(user_indices, item_indices, user_table, item_table, affine_w, affine_b):
    params = jnp.concatenate(
        [jnp.broadcast_to(affine_w.reshape(LATENT_DIM, 1), (LATENT_DIM, LANES)),
         jnp.broadcast_to(affine_b.reshape(1, 1), (1, LANES))]).astype(jnp.float32)
    out = _gmf_sc(user_indices, item_indices, user_table.T, item_table.T, params)
    return out.reshape(BATCH, 1)
